# baseline (speedup 1.0000x reference)
import jax
import jax.numpy as jnp
from jax import lax
from jax.experimental import pallas as pl
from jax.experimental.pallas import tpu as pltpu
from jax.experimental.pallas import tpu_sc as plsc

N_NODES = 50000
IN_CH = 256
NUM_G = 512
LANES = 16
BLK = 80
NBLK = N_NODES // BLK
NWORK = 32
STEPS = -(-NBLK // NWORK)
NJ = IN_CH // LANES
NQ = BLK // LANES
DMAX = 16
TRASH = NUM_G
ACC_ROWS = 528

_DNUMS = lax.GatherDimensionNumbers(
    offset_dims=(), collapsed_slice_dims=(0,), start_index_map=(0,))


def _xlane(v, idx):
    return lax.gather(v, idx[:, None], _DNUMS, (1,),
                      mode=lax.GatherScatterMode.PROMISE_IN_BOUNDS)


def _sc_pool_body(x_hbm, batch_hbm, wg_hbm, out_hbm, gout_hbm,
                  wv, idxv, sidx, xblk, sblk, gsum, zbuf, acc,
                  lsems, ssems):
    c = lax.axis_index("c")
    s = lax.axis_index("s")
    w = s * 2 + c

    pltpu.sync_copy(wg_hbm, wv)
    wr = [wv[pl.ds(LANES * j, LANES)] for j in range(NJ)]
    lane = lax.iota(jnp.int32, LANES)
    perms = [lane ^ m for m in (8, 4, 2, 1)]
    zero = jnp.zeros((LANES,), jnp.float32)
    trash = jnp.full((LANES,), TRASH, jnp.int32)
    lane0 = lane == 0

    def zrow(r, carry):
        for j in range(NJ):
            zbuf[r, pl.ds(LANES * j, LANES)] = zero
        return carry

    lax.fori_loop(0, LANES, zrow, 0)
    for q in range(NUM_G // LANES):
        gsum[pl.ds(LANES * q, LANES)] = zero
    pltpu.sync_copy(zbuf, acc.at[pl.ds(s * 33, 16)])
    pltpu.sync_copy(zbuf, acc.at[pl.ds(s * 33 + 16, 16)])
    pltpu.sync_copy(zbuf.at[pl.ds(0, 1)], acc.at[pl.ds(s * 33 + 32, 1)])
    plsc.subcore_barrier()

    def blk_of(k):
        return k * NWORK + w

    def parity(k):
        return lax.rem(k, 2), lax.rem(k, 4)

    def load_start(k):
        (buf, _), b = parity(k), blk_of(k)
        pltpu.async_copy(batch_hbm.at[pl.ds(b * BLK, BLK)], idxv.at[buf],
                         lsems.at[buf])
        pltpu.async_copy(x_hbm.at[pl.ds(b * BLK, BLK)], xblk.at[buf],
                         lsems.at[buf])

    def load_wait(k):
        (buf, _), b = parity(k), blk_of(k)
        pltpu.make_async_copy(batch_hbm.at[pl.ds(b * BLK, BLK)],
                              idxv.at[buf], lsems.at[buf]).wait()
        pltpu.make_async_copy(x_hbm.at[pl.ds(b * BLK, BLK)],
                              xblk.at[buf], lsems.at[buf]).wait()

    def scatter_start(k):
        buf, slot = parity(k)
        pltpu.async_copy(sblk.at[buf], acc.at[sidx.at[slot]], ssems.at[buf],
                         add=True)

    def scatter_wait(k):
        buf, slot = parity(k)
        pltpu.make_async_copy(sblk.at[buf], acc.at[sidx.at[slot]],
                              ssems.at[buf]).wait()

    def compute(k):
        buf, slot = parity(k)

        def gflush(prev, ae):
            pvec = lax.broadcast_in_dim(prev, (LANES,), ())
            plsc.addupdate_scatter(gsum, [pvec], ae, mask=lane0)

        def row(idxg, u, r, st):
            t, prev, sidxv, ae, av = st
            xr = [xblk[buf, r, pl.ds(LANES * j, LANES)] for j in range(NJ)]
            dot = xr[0] * wr[0]
            for j in range(1, NJ):
                dot = dot + xr[j] * wr[j]
            for p in perms:
                dot = dot + _xlane(dot, p)
            ev = jnp.exp(dot)
            cur = idxg[u]
            changed = cur != prev

            @pl.when(changed)
            def _():
                for j in range(NJ):
                    sblk[buf, t, pl.ds(LANES * j, LANES)] = av[j]
                gflush(prev, ae)

            sidxv = jnp.where(jnp.logical_and(lane == t, changed),
                              prev, sidxv)
            t = t + jnp.where(changed, 1, 0).astype(jnp.int32)

            @pl.when(t == DMAX)
            def _():
                sidx[slot, pl.ds(0, LANES)] = sidxv
                pltpu.sync_copy(sblk.at[buf], acc.at[sidx.at[slot]],
                                add=True)

            sidxv = jnp.where(t == DMAX, trash, sidxv)
            t = jnp.where(t == DMAX, 0, t)
            keep = jnp.where(changed, 0.0, 1.0).astype(jnp.float32)
            keepv = lax.broadcast_in_dim(keep, (LANES,), ())
            av = [av[j] * keepv + xr[j] * ev for j in range(NJ)]
            ae = ae * keepv + ev
            return (t, cur, sidxv, ae, av)

        def group(q, st):
            idxg = idxv[buf, pl.ds(q * LANES, LANES)]
            for u in range(LANES):
                st = row(idxg, u, q * LANES + u, st)
            return st

        idxg0 = idxv[buf, pl.ds(0, LANES)]
        st0 = (jnp.int32(0), idxg0[0], trash, zero, [zero] * NJ)
        st = lax.fori_loop(0, NQ, group, st0)
        t, prev, sidxv, ae, av = st
        for j in range(NJ):
            sblk[buf, t, pl.ds(LANES * j, LANES)] = av[j]
        gflush(prev, ae)
        sidxv = jnp.where(lane == t, prev, sidxv)
        sidx[slot, pl.ds(0, LANES)] = sidxv

    nb = (NBLK - w + NWORK - 1) // NWORK

    load_start(0)

    def step_body(k, carry):
        pl.when(blk_of(k + 1) < NBLK)(lambda: load_start(k + 1))
        pl.when(jnp.logical_and(k >= 2, k - 2 < nb))(
            lambda: scatter_wait(k - 2))

        @pl.when(blk_of(k) < NBLK)
        def _():
            load_wait(k)
            compute(k)
            scatter_start(k)

        return carry

    lax.fori_loop(0, STEPS, step_body, 0)
    pl.when(nb >= STEPS)(lambda: scatter_wait(nb - 2))
    pl.when(nb >= STEPS - 1)(lambda: scatter_wait(nb - 1))

    plsc.subcore_barrier()
    pltpu.sync_copy(acc.at[pl.ds(s * 32, 32)], out_hbm.at[c, pl.ds(s * 32, 32)])
    pltpu.sync_copy(gsum, gout_hbm.at[c, s])


def _finish_body(p_ref, g_ref, w_ref, b_ref, o_ref):
    a = p_ref[0] + p_ref[1]
    gs = jnp.sum(g_ref[...], axis=(0, 1))
    gsc = jnp.transpose(gs.reshape(1, NUM_G))
    nonempty = gsc > 0.0
    inv = jnp.where(nonempty, 1.0 / jnp.where(nonempty, gsc, 1.0), 0.0)
    pooled = a * inv
    out = jnp.dot(pooled, w_ref[...], preferred_element_type=jnp.float32)
    o_ref[...] = out + jnp.where(nonempty, b_ref[...], 0.0)


def kernel(x, batch, W_gate, b_gate, W_nn, b_nn):
    del b_gate
    batch32 = batch.astype(jnp.int32)
    wg = W_gate.reshape(IN_CH)

    mesh = plsc.VectorSubcoreMesh(core_axis_name="c", subcore_axis_name="s")
    sc_pool = pl.kernel(
        _sc_pool_body,
        mesh=mesh,
        compiler_params=pltpu.CompilerParams(
            needs_layout_passes=False, use_tc_tiling_on_sc=False),
        out_type=(
            jax.ShapeDtypeStruct((2, NUM_G, IN_CH), jnp.float32),
            jax.ShapeDtypeStruct((2, LANES, NUM_G), jnp.float32),
        ),
        scratch_types=[
            pltpu.VMEM((IN_CH,), jnp.float32),
            pltpu.VMEM((2, BLK), jnp.int32),
            pltpu.VMEM((4, DMAX), jnp.int32),
            pltpu.VMEM((2, BLK, IN_CH), jnp.float32),
            pltpu.VMEM((2, DMAX, IN_CH), jnp.float32),
            pltpu.VMEM((NUM_G,), jnp.float32),
            pltpu.VMEM((LANES, IN_CH), jnp.float32),
            pltpu.VMEM_SHARED((ACC_ROWS, IN_CH), jnp.float32),
            pltpu.SemaphoreType.DMA((2,)),
            pltpu.SemaphoreType.DMA((2,)),
        ],
    )
    partials, gparts = sc_pool(x, batch32, wg)

    out = pl.pallas_call(
        _finish_body,
        out_shape=jax.ShapeDtypeStruct((NUM_G, 2 * IN_CH), jnp.float32),
    )(partials, gparts, W_nn, b_nn.reshape(1, 2 * IN_CH))
    return out


# speedup vs baseline: 1.2632x; 1.2632x over previous
import jax
import jax.numpy as jnp
from jax import lax
from jax.experimental import pallas as pl
from jax.experimental.pallas import tpu as pltpu
from jax.experimental.pallas import tpu_sc as plsc

N_NODES = 50000
IN_CH = 256
NUM_G = 512
LANES = 16
BLK = 80
NBLK = N_NODES // BLK
NWORK = 32
STEPS = -(-NBLK // NWORK)
NJ = IN_CH // LANES
NQ = BLK // LANES

_DNUMS = lax.GatherDimensionNumbers(
    offset_dims=(), collapsed_slice_dims=(0,), start_index_map=(0,))


def _xlane(v, idx):
    return lax.gather(v, idx[:, None], _DNUMS, (1,),
                      mode=lax.GatherScatterMode.PROMISE_IN_BOUNDS)


def _sc_pool_body(x_hbm, batch_hbm, wg_hbm, out_hbm, gout_hbm,
                  wv, idxv, xblk, sblk, estage, gsum, zbuf, acc,
                  lsems, ssems):
    c = lax.axis_index("c")
    s = lax.axis_index("s")
    w = s * 2 + c

    pltpu.sync_copy(wg_hbm, wv)
    wr = [wv[pl.ds(LANES * j, LANES)] for j in range(NJ)]
    lane = lax.iota(jnp.int32, LANES)
    perms = [lane ^ m for m in (8, 4, 2, 1)]
    zeroi = jnp.zeros((LANES,), jnp.int32)
    zero = jnp.zeros((LANES,), jnp.float32)

    def zrow(r, carry):
        for j in range(NJ):
            zbuf[r, pl.ds(LANES * j, LANES)] = zero
        return carry

    lax.fori_loop(0, LANES, zrow, 0)
    for q in range(NUM_G // LANES):
        gsum[pl.ds(LANES * q, LANES)] = zero
    pltpu.sync_copy(zbuf, acc.at[pl.ds(s * 32, 16)])
    pltpu.sync_copy(zbuf, acc.at[pl.ds(s * 32 + 16, 16)])
    plsc.subcore_barrier()

    def blk_of(k):
        return k * NWORK + w

    def parity(k):
        return lax.rem(k, 2), lax.rem(k, 4)

    HA = 48

    def load_start(k):
        (buf, slot), b = parity(k), blk_of(k)
        pltpu.async_copy(batch_hbm.at[pl.ds(b * BLK, BLK)], idxv.at[slot],
                         lsems.at[buf])
        pltpu.async_copy(x_hbm.at[pl.ds(b * BLK, BLK)], xblk.at[buf],
                         lsems.at[buf])

    def load_wait(k):
        (buf, slot), b = parity(k), blk_of(k)
        pltpu.make_async_copy(batch_hbm.at[pl.ds(b * BLK, BLK)],
                              idxv.at[slot], lsems.at[buf]).wait()
        pltpu.make_async_copy(x_hbm.at[pl.ds(b * BLK, BLK)],
                              xblk.at[buf], lsems.at[buf]).wait()

    def scatter_half(k, h):
        buf, slot = parity(k)
        r0, nr = (0, HA) if h == 0 else (HA, BLK - HA)
        return (sblk.at[buf, pl.ds(r0, nr)],
                acc.at[idxv.at[slot, pl.ds(r0, nr)]], ssems.at[buf])

    def scatter_start(k, h):
        src, dst, sem = scatter_half(k, h)
        pltpu.async_copy(src, dst, sem, add=True)

    def scatter_wait(k):
        for h in (0, 1):
            src, dst, sem = scatter_half(k, h)
            pltpu.make_async_copy(src, dst, sem).wait()

    def compute(k):
        buf, slot = parity(k)

        def row4(i, carry):
            rows = [i * 4 + u for u in range(4)]
            xrs = [[xblk[buf, r, pl.ds(LANES * j, LANES)] for j in range(NJ)]
                   for r in rows]
            tots = []
            for xr in xrs:
                dot = xr[0] * wr[0]
                for j in range(1, NJ):
                    dot = dot + xr[j] * wr[j]
                tots.append(dot)
            for p in perms:
                tots = [t + _xlane(t, p) for t in tots]
            evs = [jnp.exp(t) for t in tots]
            for r, xr, ev in zip(rows, xrs, evs):
                for j in range(NJ):
                    sblk[buf, r, pl.ds(LANES * j, LANES)] = xr[j] * ev
                estage[r, pl.ds(0, LANES)] = ev
            return carry

        lax.fori_loop(0, HA // 4, row4, 0)
        scatter_start(k, 0)
        lax.fori_loop(HA // 4, BLK // 4, row4, 0)
        scatter_start(k, 1)

        for q in range(NQ):
            eq = plsc.load_gather(estage, [lane + (LANES * q), zeroi])
            idxr = idxv[slot, pl.ds(LANES * q, LANES)]
            plsc.addupdate_scatter(gsum, [idxr], eq)

    nb = (NBLK - w + NWORK - 1) // NWORK

    load_start(0)

    def step_body(k, carry):
        pl.when(blk_of(k + 1) < NBLK)(lambda: load_start(k + 1))
        pl.when(jnp.logical_and(k >= 2, k - 2 < nb))(
            lambda: scatter_wait(k - 2))

        @pl.when(blk_of(k) < NBLK)
        def _():
            load_wait(k)
            compute(k)

        return carry

    lax.fori_loop(0, STEPS, step_body, 0)
    pl.when(nb >= STEPS)(lambda: scatter_wait(nb - 2))
    pl.when(nb >= STEPS - 1)(lambda: scatter_wait(nb - 1))

    plsc.subcore_barrier()
    pltpu.sync_copy(acc.at[pl.ds(s * 32, 32)], out_hbm.at[c, pl.ds(s * 32, 32)])
    pltpu.sync_copy(gsum, gout_hbm.at[c, s])


def _finish_body(p_ref, g_ref, w_ref, b_ref, o_ref):
    a = p_ref[0] + p_ref[1]
    gs = jnp.sum(g_ref[...], axis=(0, 1))
    gsc = jnp.transpose(gs.reshape(1, NUM_G))
    nonempty = gsc > 0.0
    inv = jnp.where(nonempty, 1.0 / jnp.where(nonempty, gsc, 1.0), 0.0)
    pooled = a * inv
    out = jnp.dot(pooled, w_ref[...], preferred_element_type=jnp.float32)
    o_ref[...] = out + jnp.where(nonempty, b_ref[...], 0.0)


def kernel(x, batch, W_gate, b_gate, W_nn, b_nn):
    del b_gate
    batch32 = batch.astype(jnp.int32)
    wg = W_gate.reshape(IN_CH)

    mesh = plsc.VectorSubcoreMesh(core_axis_name="c", subcore_axis_name="s")
    sc_pool = pl.kernel(
        _sc_pool_body,
        mesh=mesh,
        compiler_params=pltpu.CompilerParams(
            needs_layout_passes=False, use_tc_tiling_on_sc=False),
        out_type=(
            jax.ShapeDtypeStruct((2, NUM_G, IN_CH), jnp.float32),
            jax.ShapeDtypeStruct((2, LANES, NUM_G), jnp.float32),
        ),
        scratch_types=[
            pltpu.VMEM((IN_CH,), jnp.float32),
            pltpu.VMEM((4, BLK), jnp.int32),
            pltpu.VMEM((2, BLK, IN_CH), jnp.float32),
            pltpu.VMEM((2, BLK, IN_CH), jnp.float32),
            pltpu.VMEM((BLK, LANES), jnp.float32),
            pltpu.VMEM((NUM_G,), jnp.float32),
            pltpu.VMEM((LANES, IN_CH), jnp.float32),
            pltpu.VMEM_SHARED((NUM_G, IN_CH), jnp.float32),
            pltpu.SemaphoreType.DMA((2,)),
            pltpu.SemaphoreType.DMA((2,)),
        ],
    )
    partials, gparts = sc_pool(x, batch32, wg)

    out = pl.pallas_call(
        _finish_body,
        out_shape=jax.ShapeDtypeStruct((NUM_G, 2 * IN_CH), jnp.float32),
    )(partials, gparts, W_nn, b_nn.reshape(1, 2 * IN_CH))
    return out


# speedup vs baseline: 1.2705x; 1.0058x over previous
import jax
import jax.numpy as jnp
from jax import lax
from jax.experimental import pallas as pl
from jax.experimental.pallas import tpu as pltpu
from jax.experimental.pallas import tpu_sc as plsc

N_NODES = 50000
IN_CH = 256
NUM_G = 512
LANES = 16
BLK = 80
NBLK = N_NODES // BLK
NWORK = 32
STEPS = -(-NBLK // NWORK)
NJ = IN_CH // LANES
NQ = BLK // LANES

_DNUMS = lax.GatherDimensionNumbers(
    offset_dims=(), collapsed_slice_dims=(0,), start_index_map=(0,))


def _xlane(v, idx):
    return lax.gather(v, idx[:, None], _DNUMS, (1,),
                      mode=lax.GatherScatterMode.PROMISE_IN_BOUNDS)


def _sc_pool_body(x_hbm, batch_hbm, wg_hbm, out_hbm, gout_hbm,
                  wv, idxv, xblk, sblk, estage, gsum, zbuf, acc,
                  lsems, ssems):
    c = lax.axis_index("c")
    s = lax.axis_index("s")
    w = s * 2 + c

    pltpu.sync_copy(wg_hbm, wv)
    wr = [wv[pl.ds(LANES * j, LANES)] for j in range(NJ)]
    lane = lax.iota(jnp.int32, LANES)
    perms = [lane ^ m for m in (8, 4, 2, 1)]
    zeroi = jnp.zeros((LANES,), jnp.int32)
    zero = jnp.zeros((LANES,), jnp.float32)

    def zrow(r, carry):
        for j in range(NJ):
            zbuf[r, pl.ds(LANES * j, LANES)] = zero
        return carry

    lax.fori_loop(0, LANES, zrow, 0)
    for q in range(NUM_G // LANES):
        gsum[pl.ds(LANES * q, LANES)] = zero
    pltpu.sync_copy(zbuf, acc.at[pl.ds(s * 32, 16)])
    pltpu.sync_copy(zbuf, acc.at[pl.ds(s * 32 + 16, 16)])
    plsc.subcore_barrier()

    def blk_of(k):
        return k * NWORK + w

    def parity(k):
        return lax.rem(k, 2), lax.rem(k, 4)

    def load_start(k):
        (buf, slot), b = parity(k), blk_of(k)
        pltpu.async_copy(batch_hbm.at[pl.ds(b * BLK, BLK)], idxv.at[slot],
                         lsems.at[buf])
        pltpu.async_copy(x_hbm.at[pl.ds(b * BLK, BLK)], xblk.at[buf],
                         lsems.at[buf])

    def load_wait(k):
        (buf, slot), b = parity(k), blk_of(k)
        pltpu.make_async_copy(batch_hbm.at[pl.ds(b * BLK, BLK)],
                              idxv.at[slot], lsems.at[buf]).wait()
        pltpu.make_async_copy(x_hbm.at[pl.ds(b * BLK, BLK)],
                              xblk.at[buf], lsems.at[buf]).wait()

    def scatter_start(k):
        buf, slot = parity(k)
        pltpu.async_copy(sblk.at[buf], acc.at[idxv.at[slot]], ssems.at[buf],
                         add=True)

    def scatter_wait(k):
        buf, slot = parity(k)
        pltpu.make_async_copy(sblk.at[buf], acc.at[idxv.at[slot]],
                              ssems.at[buf]).wait()

    def compute(k):
        buf, slot = parity(k)

        def row4(i, carry):
            rows = [i * 4 + u for u in range(4)]
            xrs = [[xblk[buf, r, pl.ds(LANES * j, LANES)] for j in range(NJ)]
                   for r in rows]
            tots = []
            for xr in xrs:
                dot = xr[0] * wr[0]
                for j in range(1, NJ):
                    dot = dot + xr[j] * wr[j]
                tots.append(dot)
            for p in perms:
                tots = [t + _xlane(t, p) for t in tots]
            evs = [jnp.exp(t) for t in tots]
            for r, xr, ev in zip(rows, xrs, evs):
                for j in range(NJ):
                    sblk[buf, r, pl.ds(LANES * j, LANES)] = xr[j] * ev
                estage[r, pl.ds(0, LANES)] = ev
            return carry

        lax.fori_loop(0, BLK // 4, row4, 0)

        for q in range(NQ):
            eq = plsc.load_gather(estage, [lane + (LANES * q), zeroi])
            idxr = idxv[slot, pl.ds(LANES * q, LANES)]
            plsc.addupdate_scatter(gsum, [idxr], eq)

    nb = (NBLK - w + NWORK - 1) // NWORK

    load_start(0)

    def step_body(k, carry):
        pl.when(blk_of(k + 1) < NBLK)(lambda: load_start(k + 1))
        pl.when(jnp.logical_and(k >= 2, k - 2 < nb))(
            lambda: scatter_wait(k - 2))

        @pl.when(blk_of(k) < NBLK)
        def _():
            load_wait(k)
            compute(k)
            scatter_start(k)

        return carry

    lax.fori_loop(0, STEPS, step_body, 0)
    pl.when(nb >= STEPS)(lambda: scatter_wait(nb - 2))
    pl.when(nb >= STEPS - 1)(lambda: scatter_wait(nb - 1))

    plsc.subcore_barrier()
    pltpu.sync_copy(acc.at[pl.ds(s * 32, 32)], out_hbm.at[c, pl.ds(s * 32, 32)])
    pltpu.sync_copy(gsum, gout_hbm.at[c, s])


def _finish_body(p_ref, g_ref, w_ref, b_ref, o_ref):
    a = p_ref[0] + p_ref[1]
    gs = jnp.sum(g_ref[...], axis=(0, 1))
    gsc = jnp.transpose(gs.reshape(1, NUM_G))
    nonempty = gsc > 0.0
    inv = jnp.where(nonempty, 1.0 / jnp.where(nonempty, gsc, 1.0), 0.0)
    pooled = a * inv
    out = jnp.dot(pooled, w_ref[...], preferred_element_type=jnp.float32)
    o_ref[...] = out + jnp.where(nonempty, b_ref[...], 0.0)


def kernel(x, batch, W_gate, b_gate, W_nn, b_nn):
    del b_gate
    batch32 = batch.astype(jnp.int32)
    wg = W_gate.reshape(IN_CH)

    mesh = plsc.VectorSubcoreMesh(core_axis_name="c", subcore_axis_name="s")
    sc_pool = pl.kernel(
        _sc_pool_body,
        mesh=mesh,
        compiler_params=pltpu.CompilerParams(
            needs_layout_passes=False, use_tc_tiling_on_sc=False),
        out_type=(
            jax.ShapeDtypeStruct((2, NUM_G, IN_CH), jnp.float32),
            jax.ShapeDtypeStruct((2, LANES, NUM_G), jnp.float32),
        ),
        scratch_types=[
            pltpu.VMEM((IN_CH,), jnp.float32),
            pltpu.VMEM((4, BLK), jnp.int32),
            pltpu.VMEM((2, BLK, IN_CH), jnp.float32),
            pltpu.VMEM((2, BLK, IN_CH), jnp.float32),
            pltpu.VMEM((BLK, LANES), jnp.float32),
            pltpu.VMEM((NUM_G,), jnp.float32),
            pltpu.VMEM((LANES, IN_CH), jnp.float32),
            pltpu.VMEM_SHARED((NUM_G, IN_CH), jnp.float32),
            pltpu.SemaphoreType.DMA((2,)),
            pltpu.SemaphoreType.DMA((2,)),
        ],
    )
    partials, gparts = sc_pool(x, batch32, wg)

    out = pl.pallas_call(
        _finish_body,
        out_shape=jax.ShapeDtypeStruct((NUM_G, 2 * IN_CH), jnp.float32),
    )(partials, gparts, W_nn, b_nn.reshape(1, 2 * IN_CH))
    return out
